# Initial kernel scaffold; baseline (speedup 1.0000x reference)
import functools

import jax
import jax.numpy as jnp
from jax import lax
from jax.experimental import pallas as pl
from jax.experimental.pallas import tpu as pltpu
from jax.experimental.pallas import tpu_sc as plsc

VOCAB = 1000000
DIM = 64
B = 16384
L = 50

_N_FLAT = B * L

_info = plsc.get_sparse_core_info()
_NC, _NS = _info.num_cores, _info.num_subcores
_NW = _NC * _NS
_PER_W = _N_FLAT // _NW
_CHUNK = 1024
_N_CHUNKS = _PER_W // _CHUNK


def _make_gather():
    mesh = plsc.VectorSubcoreMesh(core_axis_name="c", subcore_axis_name="s")

    @functools.partial(
        pl.kernel,
        mesh=mesh,
        out_type=jax.ShapeDtypeStruct((_N_FLAT, DIM), jnp.float32),
        scratch_types=[
            pltpu.VMEM((_CHUNK,), jnp.int32),
            pltpu.VMEM((_CHUNK, DIM), jnp.float32),
            pltpu.SemaphoreType.DMA,
        ],
    )
    def gather_kernel(idx_hbm, table_hbm, out_hbm, idx_v, rows_v, sem):
        wid = lax.axis_index("s") * _NC + lax.axis_index("c")
        w_base = wid * _PER_W

        def body(j, carry):
            base = w_base + j * _CHUNK
            pltpu.sync_copy(idx_hbm.at[pl.ds(base, _CHUNK)], idx_v)
            pltpu.async_copy(table_hbm.at[idx_v], rows_v, sem).wait()
            pltpu.sync_copy(rows_v, out_hbm.at[pl.ds(base, _CHUNK)])
            return carry

        lax.fori_loop(0, _N_CHUNKS, body, 0)

    return gather_kernel


_gather = _make_gather()


def kernel(x, table):
    flat_idx = x.reshape(_N_FLAT).astype(jnp.int32)
    out = _gather(flat_idx, table)
    return out.reshape(B, L, DIM)


# speedup vs baseline: 1.8422x; 1.8422x over previous
import functools

import jax
import jax.numpy as jnp
from jax import lax
from jax.experimental import pallas as pl
from jax.experimental.pallas import tpu as pltpu
from jax.experimental.pallas import tpu_sc as plsc

VOCAB = 1000000
DIM = 64
B = 16384
L = 50

_N_FLAT = B * L

_info = plsc.get_sparse_core_info()
_NC, _NS = _info.num_cores, _info.num_subcores
_NW = _NC * _NS
_PER_W = _N_FLAT // _NW
_CHUNK = 1024
_N_CHUNKS = _PER_W // _CHUNK


def _make_gather():
    mesh = plsc.VectorSubcoreMesh(core_axis_name="c", subcore_axis_name="s")

    @functools.partial(
        pl.kernel,
        mesh=mesh,
        out_type=jax.ShapeDtypeStruct((_N_FLAT, DIM), jnp.float32),
        scratch_types=[
            pltpu.VMEM((_CHUNK,), jnp.int32),
            pltpu.VMEM((_CHUNK, DIM), jnp.float32),
            pltpu.SemaphoreType.DMA,
        ],
        compiler_params=pltpu.CompilerParams(use_tc_tiling_on_sc=False),
    )
    def gather_kernel(idx_hbm, table_hbm, out_hbm, idx_v, rows_v, sem):
        wid = lax.axis_index("s") * _NC + lax.axis_index("c")
        w_base = wid * _PER_W

        def body(j, carry):
            base = w_base + j * _CHUNK
            pltpu.sync_copy(idx_hbm.at[pl.ds(base, _CHUNK)], idx_v)
            pltpu.async_copy(table_hbm.at[idx_v], rows_v, sem).wait()
            pltpu.sync_copy(rows_v, out_hbm.at[pl.ds(base, _CHUNK)])
            return carry

        lax.fori_loop(0, _N_CHUNKS, body, 0)

    return gather_kernel


_gather = _make_gather()


def kernel(x, table):
    flat_idx = x.reshape(_N_FLAT).astype(jnp.int32)
    out = _gather(flat_idx, table)
    return out.reshape(B, L, DIM)


# speedup vs baseline: 1.8660x; 1.0129x over previous
import functools

import jax
import jax.numpy as jnp
from jax import lax
from jax.experimental import pallas as pl
from jax.experimental.pallas import tpu as pltpu
from jax.experimental.pallas import tpu_sc as plsc

VOCAB = 1000000
DIM = 64
B = 16384
L = 50

_N_FLAT = B * L

_info = plsc.get_sparse_core_info()
_NC, _NS = _info.num_cores, _info.num_subcores
_NW = _NC * _NS
_PER_W = _N_FLAT // _NW
_NBUF = 4
_CHUNK = 400
_N_CHUNKS = _PER_W // _CHUNK
_N_GROUPS = _N_CHUNKS // _NBUF


def _make_gather():
    mesh = plsc.VectorSubcoreMesh(core_axis_name="c", subcore_axis_name="s")

    @functools.partial(
        pl.kernel,
        mesh=mesh,
        out_type=jax.ShapeDtypeStruct((_N_FLAT, DIM), jnp.float32),
        scratch_types=[
            pltpu.VMEM((_PER_W,), jnp.int32),
            pltpu.VMEM((_NBUF, _CHUNK, DIM), jnp.float32),
            pltpu.SemaphoreType.DMA((_NBUF,)),
            pltpu.SemaphoreType.DMA((_NBUF,)),
        ],
        compiler_params=pltpu.CompilerParams(use_tc_tiling_on_sc=False),
    )
    def gather_kernel(idx_hbm, table_hbm, out_hbm, idx_v, rows_v, g_sem, s_sem):
        wid = lax.axis_index("s") * _NC + lax.axis_index("c")
        w_base = wid * _PER_W

        pltpu.sync_copy(idx_hbm.at[pl.ds(w_base, _PER_W)], idx_v)

        def start_gather(j, b):
            pltpu.async_copy(
                table_hbm.at[idx_v.at[pl.ds(j * _CHUNK, _CHUNK)]],
                rows_v.at[b],
                g_sem.at[b],
            )

        def drain_and_scatter(j, b):
            pltpu.make_async_copy(
                table_hbm.at[idx_v.at[pl.ds(j * _CHUNK, _CHUNK)]],
                rows_v.at[b],
                g_sem.at[b],
            ).wait()
            pltpu.async_copy(
                rows_v.at[b],
                out_hbm.at[pl.ds(w_base + j * _CHUNK, _CHUNK)],
                s_sem.at[b],
            )

        def wait_scatter(j, b):
            pltpu.make_async_copy(
                rows_v.at[b],
                out_hbm.at[pl.ds(w_base + j * _CHUNK, _CHUNK)],
                s_sem.at[b],
            ).wait()

        for b in range(_NBUF):
            start_gather(b, b)
        for b in range(_NBUF):
            drain_and_scatter(b, b)

        def body(p, carry):
            for b in range(_NBUF):
                j = p * _NBUF + b
                wait_scatter(j - _NBUF, b)
                start_gather(j, b)
            for b in range(_NBUF):
                j = p * _NBUF + b
                drain_and_scatter(j, b)
            return carry

        lax.fori_loop(1, _N_GROUPS, body, 0)

        for b in range(_NBUF):
            wait_scatter((_N_GROUPS - 1) * _NBUF + b, b)

    return gather_kernel


_gather = _make_gather()


def kernel(x, table):
    flat_idx = x.reshape(_N_FLAT).astype(jnp.int32)
    out = _gather(flat_idx, table)
    return out.reshape(B, L, DIM)
